# Initial kernel scaffold; baseline (speedup 1.0000x reference)
import jax
import jax.numpy as jnp
from jax.experimental import pallas as pl


def kernel(x, edge_index, edge_attr, W1, We1, as1, ad1, ae1, b1, Wl1, bl1, W2, We2, as2, ad2, ae2, b2, Wl2, bl2, W3, We3, as3, ad3, ae3, b3, Wl3, bl3):
    raise NotImplementedError("write your pallas kernel here")


# speedup vs baseline: 1.1369x; 1.1369x over previous
import jax
import jax.numpy as jnp
from jax.experimental import pallas as pl


def _layer(x, src, dst, ea, W, We, as_, ad_, ae_, b, heads, ch, concat, deg):
    n, F = x.shape
    E = src.shape[0]
    ED = ea.shape[1]
    vs = (W.reshape(F, heads, ch) * as_[None]).sum(-1)
    vd = (W.reshape(F, heads, ch) * ad_[None]).sum(-1)
    ve = (We.reshape(ED, heads, ch) * ae_[None]).sum(-1)
    asn = x @ vs
    adn = x @ vd
    aee = ea @ ve
    ae_sum = jax.ops.segment_sum(aee, dst, num_segments=n)
    ae_loop = ae_sum / jnp.maximum(deg, 1.0)[:, None]
    exl = jnp.exp(jax.nn.leaky_relu(asn + adn + ae_loop, 0.2))
    exe = jnp.exp(jax.nn.leaky_relu(asn[src] + adn[dst] + aee, 0.2))
    denom = jax.ops.segment_sum(exe, dst, num_segments=n) + exl
    xh = x @ W
    msg = jax.ops.segment_sum(xh[src].reshape(E, heads, ch) * exe[:, :, None],
                              dst, num_segments=n)
    wsum = msg + exl[:, :, None] * xh.reshape(n, heads, ch)
    out = wsum / denom[:, :, None]
    if concat:
        out = out.reshape(n, heads * ch)
    else:
        out = out.mean(axis=1)
    return out + b


def kernel(x, edge_index, edge_attr, W1, We1, as1, ad1, ae1, b1, Wl1, bl1, W2, We2, as2, ad2, ae2, b2, Wl2, bl2, W3, We3, as3, ad3, ae3, b3, Wl3, bl3):
    src, dst = edge_index[0], edge_index[1]
    n = x.shape[0]
    deg = jax.ops.segment_sum(jnp.ones(src.shape[0], jnp.float32), dst, num_segments=n)
    h = jax.nn.elu(_layer(x, src, dst, edge_attr, W1, We1, as1, ad1, ae1, b1, 4, 256, True, deg) + x @ Wl1 + bl1)
    h = jax.nn.elu(_layer(h, src, dst, edge_attr, W2, We2, as2, ad2, ae2, b2, 4, 256, True, deg) + h @ Wl2 + bl2)
    out = _layer(h, src, dst, edge_attr, W3, We3, as3, ad3, ae3, b3, 6, 40, False, deg) + h @ Wl3 + bl3
    return out


# speedup vs baseline: 13.7819x; 12.1228x over previous
import functools

import jax
import jax.numpy as jnp
from jax import lax
from jax.experimental import pallas as pl
from jax.experimental.pallas import tpu as pltpu
from jax.experimental.pallas import tpu_sc as plsc

F32 = jnp.float32
N = 10000
NP = 10240
E = 320000
BN = 512
NB = NP // BN
NC, NS = 2, 16
NW = NC * NS
K = 128
NCHUNK = E // K
NCHT = (NCHUNK + NW - 1) // NW

_mesh = plsc.VectorSubcoreMesh(core_axis_name="c", subcore_axis_name="s",
                               num_cores=NC, num_subcores=NS)
_sc_params = pltpu.CompilerParams(needs_layout_passes=False)


def _prologue(x, Wv, Wl, bvec, Vsd, VH, lin_blocked):
    F = x.shape[1]
    LW = Wl.shape[1]
    AW = Vsd.shape[1]
    lwb = 128 if lin_blocked else LW

    def body(x_ref, wv_ref, wl_ref, b_ref, vsd_ref, xh_o, lin_o, aa_o):
        vh = pl.program_id(1)
        xb = x_ref[...]
        xh_o[0] = jnp.dot(xb, wv_ref[0], preferred_element_type=F32)
        if lin_blocked:
            lin_o[...] = jnp.dot(xb, wl_ref[...],
                                 preferred_element_type=F32) + b_ref[...]
        else:
            @pl.when(vh == 0)
            def _():
                lin_o[...] = jnp.dot(xb, wl_ref[...],
                                     preferred_element_type=F32) + b_ref[...]

        @pl.when(vh == 0)
        def _():
            aa_o[...] = jnp.dot(xb, vsd_ref[...], preferred_element_type=F32)

    if lin_blocked:
        lin_ix = lambda nb, vh: (nb, vh)
        wl_ix = lambda nb, vh: (0, vh)
    else:
        lin_ix = lambda nb, vh: (nb, 0)
        wl_ix = lambda nb, vh: (0, 0)

    return pl.pallas_call(
        body,
        grid=(NB, VH),
        in_specs=[
            pl.BlockSpec((BN, F), lambda nb, vh: (nb, 0)),
            pl.BlockSpec((1, F, 128), lambda nb, vh: (vh, 0, 0)),
            pl.BlockSpec((F, lwb), wl_ix),
            pl.BlockSpec((1, lwb), wl_ix),
            pl.BlockSpec((F, AW), lambda nb, vh: (0, 0)),
        ],
        out_specs=[
            pl.BlockSpec((1, BN, 128), lambda nb, vh: (vh, nb, 0)),
            pl.BlockSpec((BN, lwb), lin_ix),
            pl.BlockSpec((BN, AW), lambda nb, vh: (nb, 0)),
        ],
        out_shape=[
            jax.ShapeDtypeStruct((VH, NP, 128), F32),
            jax.ShapeDtypeStruct((NP, LW), F32),
            jax.ShapeDtypeStruct((NP, AW), F32),
        ],
    )(x, Wv, Wl, bvec, Vsd)


def _nodeprep(asn, adn, aesum, deg, denE):
    H = asn.shape[1]

    def body(asn_r, adn_r, aes_r, deg_r, den_r, nd_o):
        degc = jnp.maximum(deg_r[...], 1.0)
        s = asn_r[...] + adn_r[...] + aes_r[...] / degc
        al = jnp.where(s > 0, s, 0.2 * s)
        exl = jnp.exp(al)
        den = den_r[...] + exl
        pad = jnp.zeros((BN, 8 - H), F32)
        nd_o[...] = jnp.concatenate([exl, pad, den, pad], axis=1)

    return pl.pallas_call(
        body,
        grid=(NB,),
        in_specs=[pl.BlockSpec((BN, H), lambda nb: (nb, 0))] * 3
        + [pl.BlockSpec((BN, 1), lambda nb: (nb, 0)),
           pl.BlockSpec((BN, H), lambda nb: (nb, 0))],
        out_specs=pl.BlockSpec((BN, 16), lambda nb: (nb, 0)),
        out_shape=jax.ShapeDtypeStruct((NP, 16), F32),
    )(asn, adn, aesum, deg, denE)


def _sum32(acc, C):
    def body(a_ref, o_ref):
        o_ref[...] = jnp.sum(a_ref[...], axis=0)

    return pl.pallas_call(
        body,
        grid=(NB,),
        in_specs=[pl.BlockSpec((NW, BN, C), lambda nb: (0, nb, 0))],
        out_specs=pl.BlockSpec((BN, C), lambda nb: (nb, 0)),
        out_shape=jax.ShapeDtypeStruct((NP, C), F32),
    )(acc)


def _epi12(msg, xh, nd, lin):
    def body(msg_ref, xh_ref, nd_ref, lin_ref, o_ref):
        vh = pl.program_id(1)
        h = vh // 2
        nd_blk = nd_ref[...]
        li = lax.broadcasted_iota(jnp.int32, (1, 16), 1)
        exl = jnp.sum(jnp.where(li == h, nd_blk, 0.0), axis=1, keepdims=True)
        den = jnp.sum(jnp.where(li == 8 + h, nd_blk, 0.0), axis=1,
                      keepdims=True)
        m = msg_ref[0, 0] + msg_ref[1, 0]
        v = (m + exl * xh_ref[0]) / den + lin_ref[...]
        o_ref[...] = jnp.where(v > 0, v, jnp.exp(jnp.minimum(v, 0.0)) - 1.0)

    return pl.pallas_call(
        body,
        grid=(NB, 8),
        in_specs=[
            pl.BlockSpec((2, 1, BN, 128), lambda nb, vh: (0, vh, nb, 0)),
            pl.BlockSpec((1, BN, 128), lambda nb, vh: (vh, nb, 0)),
            pl.BlockSpec((BN, 16), lambda nb, vh: (nb, 0)),
            pl.BlockSpec((BN, 128), lambda nb, vh: (nb, vh)),
        ],
        out_specs=pl.BlockSpec((BN, 128), lambda nb, vh: (nb, vh)),
        out_shape=jax.ShapeDtypeStruct((NP, 1024), F32),
    )(msg, xh, nd, lin)


def _epi3(msg, xh, nd, lin):
    def body(msg_ref, xh_ref, nd_ref, lin_ref, o_ref):
        acc = jnp.zeros((BN, 64), F32)
        for h in range(6):
            lo = (h % 2) * 64
            m = msg_ref[0, h // 2, :, lo:lo + 64] + \
                msg_ref[1, h // 2, :, lo:lo + 64]
            exl = nd_ref[:, h:h + 1]
            den = nd_ref[:, 8 + h:9 + h]
            acc = acc + (m + exl * xh_ref[h // 2, :, lo:lo + 64]) / den
        o_ref[...] = acc[:, :40] / 6.0 + lin_ref[:, :40]

    return pl.pallas_call(
        body,
        grid=(NB,),
        in_specs=[
            pl.BlockSpec((2, 3, BN, 128), lambda nb: (0, 0, nb, 0)),
            pl.BlockSpec((3, BN, 128), lambda nb: (0, nb, 0)),
            pl.BlockSpec((BN, 16), lambda nb: (nb, 0)),
            pl.BlockSpec((BN, 64), lambda nb: (nb, 0)),
        ],
        out_specs=pl.BlockSpec((BN, 40), lambda nb: (nb, 0)),
        out_shape=jax.ShapeDtypeStruct((N, 40), F32),
    )(msg, xh, nd, lin)


def _zero_vmem(ref, nslices):
    z = jnp.zeros((16,), F32)

    def zbody(i, carry):
        for t in range(16):
            ref[pl.ds((i * 16 + t) * 16, 16)] = z
        return carry

    lax.fori_loop(0, nslices // 16, zbody, 0)


def _edge_accum(cols):
    @functools.partial(
        pl.kernel,
        out_type=jax.ShapeDtypeStruct((NW, NP * 8), F32),
        mesh=_mesh,
        compiler_params=_sc_params,
        scratch_types=[
            pltpu.VMEM((NP * 8,), F32),
            pltpu.VMEM((K,), jnp.int32),
            pltpu.VMEM((6, K), F32),
            pltpu.VMEM((6, 16), F32),
        ],
    )
    def k(dsth, ea0, ea1, ea2, ea3, ea4, ea5, vek, acc_o, acc, dstv, eabuf,
          vebuf):
        c = lax.axis_index("c")
        s = lax.axis_index("s")
        wid = c * NS + s
        iota = lax.iota(jnp.int32, 16)
        _zero_vmem(acc, NP * 8 // 16)
        pltpu.sync_copy(vek, vebuf)
        eas = (ea0, ea1, ea2, ea3, ea4, ea5)

        def chunk(j, carry):
            m = j * NW + wid

            @pl.when(m < NCHUNK)
            def _():
                base = m * K
                pltpu.sync_copy(dsth.at[pl.ds(base, K)], dstv)
                for kk in range(6):
                    pltpu.sync_copy(eas[kk].at[pl.ds(base, K)],
                                    eabuf.at[kk])
                for g in range(K // 16):
                    nid = dstv[pl.ds(g * 16, 16)]
                    eak = [eabuf[kk, pl.ds(g * 16, 16)] for kk in range(6)]
                    for col, hv in cols:
                        if hv is None:
                            val = jnp.full((16,), 1.0, F32)
                        else:
                            ver = [vebuf[kk, :] for kk in range(6)]
                            val = eak[0] * ver[0][hv]
                            for kk in range(1, 6):
                                val = val + eak[kk] * ver[kk][hv]
                        plsc.addupdate_scatter(
                            acc, [nid * 8 + col], val)
            return carry

        lax.fori_loop(0, NCHT, chunk, 0)
        pltpu.sync_copy(acc, acc_o.at[wid])

    return k


def _edge_exp(Ht, asn_col, adn_col, ve_col0):
    @functools.partial(
        pl.kernel,
        out_type=[jax.ShapeDtypeStruct((E,), F32) for _ in range(Ht)]
        + [jax.ShapeDtypeStruct((NW, NP * 4), F32)],
        mesh=_mesh,
        compiler_params=_sc_params,
        scratch_types=[
            pltpu.VMEM((NP * 8,), F32),
            pltpu.VMEM((NP * 4,), F32),
            pltpu.VMEM((K,), jnp.int32),
            pltpu.VMEM((K,), jnp.int32),
            pltpu.VMEM((6, K), F32),
            pltpu.VMEM((6, 16), F32),
            pltpu.VMEM((8, K), F32),
        ],
    )
    def k(srch, dsth, ea0, ea1, ea2, ea3, ea4, ea5, vek, tblh, *rest):
        exe_os = rest[:Ht]
        acc_o = rest[Ht]
        tblv, acc, srcv, dstv, eabuf, vebuf, ebuf = rest[Ht + 1:Ht + 8]
        c = lax.axis_index("c")
        s = lax.axis_index("s")
        wid = c * NS + s
        _zero_vmem(acc, NP * 4 // 16)
        pltpu.sync_copy(vek, vebuf)
        pltpu.sync_copy(tblh, tblv)
        eas = (ea0, ea1, ea2, ea3, ea4, ea5)

        def chunk(j, carry):
            m = j * NW + wid

            @pl.when(m < NCHUNK)
            def _():
                base = m * K
                pltpu.sync_copy(srch.at[pl.ds(base, K)], srcv)
                pltpu.sync_copy(dsth.at[pl.ds(base, K)], dstv)
                for kk in range(6):
                    pltpu.sync_copy(eas[kk].at[pl.ds(base, K)],
                                    eabuf.at[kk])
                for g in range(K // 16):
                    nid_s = srcv[pl.ds(g * 16, 16)]
                    nid_d = dstv[pl.ds(g * 16, 16)]
                    eak = [eabuf[kk, pl.ds(g * 16, 16)] for kk in range(6)]
                    ver = [vebuf[kk, :] for kk in range(6)]
                    for h in range(Ht):
                        asn = plsc.load_gather(
                            tblv, [nid_s * 8 + (asn_col + h)])
                        adn = plsc.load_gather(
                            tblv, [nid_d * 8 + (adn_col + h)])
                        aee = eak[0] * ver[0][ve_col0 + h]
                        for kk in range(1, 6):
                            aee = aee + eak[kk] * ver[kk][ve_col0 + h]
                        aa = asn + adn + aee
                        al = jnp.where(aa > 0, aa, aa * 0.2)
                        ex = jnp.exp(al)
                        ebuf[h, pl.ds(g * 16, 16)] = ex
                        plsc.addupdate_scatter(acc, [nid_d * 4 + h], ex)
                for h in range(Ht):
                    pltpu.sync_copy(ebuf.at[h],
                                    exe_os[h].at[pl.ds(base, K)])
            return carry

        lax.fori_loop(0, NCHT, chunk, 0)
        pltpu.sync_copy(acc, acc_o.at[wid])

    return k


def _spmm(srch, dsth, exe_h, xh_vh, zrow):
    RPT = NP // NS

    @functools.partial(
        pl.kernel,
        out_type=jax.ShapeDtypeStruct((2, NP, 128), F32),
        mesh=_mesh,
        scratch_types=[
            pltpu.VMEM_SHARED((NP, 128), F32),
            pltpu.VMEM((K,), jnp.int32),
            pltpu.VMEM((K,), jnp.int32),
            pltpu.VMEM((K,), F32),
            pltpu.VMEM((K, 128), F32),
            pltpu.SemaphoreType.DMA,
        ],
    )
    def k(srch, dsth, exeh, xhv, zr, msg_o, acc, srcv, dstv, wv, rows, sem):
        c = lax.axis_index("c")
        s = lax.axis_index("s")
        pltpu.sync_copy(zr, acc.at[pl.ds(s * RPT, RPT)])
        plsc.subcore_barrier()
        half = NCHUNK // 2

        def chunk(j, carry):
            m = c * half + j * NS + s

            @pl.when(m < (c + 1) * half)
            def _():
                base = m * K
                pltpu.sync_copy(srch.at[pl.ds(base, K)], srcv)
                pltpu.sync_copy(dsth.at[pl.ds(base, K)], dstv)
                pltpu.sync_copy(exeh.at[pl.ds(base, K)], wv)
                pltpu.async_copy(xhv.at[srcv], rows, sem).wait()

                def egroup(g, carry2):
                    wsl = wv[pl.ds(g * 16, 16)]
                    for kk in range(16):
                        e = g * 16 + kk
                        w = wsl[kk]
                        for q in range(8):
                            sl = pl.ds(q * 16, 16)
                            rows[e, sl] = rows[e, sl] * w
                    return carry2

                lax.fori_loop(0, K // 16, egroup, 0)
                pltpu.sync_copy(rows, acc.at[dstv], add=True)
            return carry

        lax.fori_loop(0, (half + NS - 1) // NS, chunk, 0)
        plsc.subcore_barrier()
        pltpu.sync_copy(acc.at[pl.ds(s * RPT, RPT)],
                        msg_o.at[c, pl.ds(s * RPT, RPT), :])

    return k(srch, dsth, exe_h, xh_vh, zrow)


def _fold(W, att):
    F = W.shape[0]
    H, C = att.shape
    return (W.reshape(F, H, C) * att[None]).sum(-1)


def _vep(We, ae_, col0):
    out = jnp.zeros((6, 16), F32)
    return out.at[:, col0:col0 + ae_.shape[0]].set(_fold(We, ae_))


def kernel(x, edge_index, edge_attr, W1, We1, as1, ad1, ae1, b1, Wl1, bl1,
           W2, We2, as2, ad2, ae2, b2, Wl2, bl2,
           W3, We3, as3, ad3, ae3, b3, Wl3, bl3):
    srch = edge_index[0]
    dsth = edge_index[1]
    eat = [edge_attr[:, kk] for kk in range(6)]
    zr128 = jnp.zeros((NP // NS, 128), F32)

    veA = jnp.zeros((6, 16), F32)
    veA = veA.at[:, 0:4].set(_fold(We1, ae1)).at[:, 4:8].set(_fold(We2, ae2))
    veB = jnp.zeros((6, 16), F32)
    veB = veB.at[:, 0:6].set(_fold(We3, ae3))

    accA12 = _edge_accum([(h, h) for h in range(8)])(dsth, *eat, veA)
    accA3 = _edge_accum([(h, h) for h in range(6)] + [(7, None)])(
        dsth, *eat, veB)
    sumA12 = _sum32(accA12.reshape(NW, NP, 8), 8)
    sumA3 = _sum32(accA3.reshape(NW, NP, 8), 8)
    deg = sumA3[:, 7:8]

    def layer12(x_in, W, Wl, b, bl, as_, ad_, ve_col0):
        F = W.shape[0]
        bias = (b + bl).reshape(1, 1024)
        vs = _fold(W, as_)
        vd = _fold(W, ad_)
        vsd = jnp.concatenate([vs, vd], axis=1)
        Wv = W.reshape(F, 8, 128).transpose(1, 0, 2)
        xh, lin, aa = _prologue(x_in, Wv, Wl, bias, vsd, 8, True)
        veK = veA
        outs = _edge_exp(4, 0, 4, ve_col0)(srch, dsth, *eat, veK,
                                           aa.reshape(-1))
        exes, accB = outs[:4], outs[4]
        denE = _sum32(accB.reshape(NW, NP, 4), 4)
        msgs = []
        for vh in range(8):
            xh_vh = xh[vh]
            ex_vh = exes[vh // 2]
            msgs.append(_spmm(srch, dsth, ex_vh, xh_vh, zr128))
        msg = jnp.stack(msgs, axis=1)
        nd = _nodeprep(aa[:, 0:4], aa[:, 4:8], sumA12[:, ve_col0:ve_col0 + 4],
                       deg, denE)
        return _epi12(msg, xh, nd, lin)

    h1 = layer12(x, W1, Wl1, b1, bl1, as1, ad1, 0)
    h2 = layer12(h1, W2, Wl2, b2, bl2, as2, ad2, 4)

    W3v = jnp.pad(W3.reshape(1024, 6, 40),
                  ((0, 0), (0, 0), (0, 24))).reshape(1024, 3, 128)
    W3v = W3v.transpose(1, 0, 2)
    Wl3p = jnp.pad(Wl3, ((0, 0), (0, 24)))
    bias3 = jnp.pad(b3 + bl3, (0, 24)).reshape(1, 64)
    vs3 = _fold(W3, as3)
    vd3 = _fold(W3, ad3)
    vsd3 = jnp.concatenate([vs3[:, 0:3], vd3[:, 0:3], jnp.zeros_like(vs3[:, :1]), jnp.zeros_like(vs3[:, :1]),
                            vs3[:, 3:6], vd3[:, 3:6], jnp.zeros_like(vs3[:, :1]), jnp.zeros_like(vs3[:, :1])], axis=1)
    xh3, lin3, aa3 = _prologue(h2, W3v, Wl3p, bias3, vsd3, 3, False)
    tbl3a = aa3[:, 0:8]
    tbl3b = aa3[:, 8:16]
    outsA = _edge_exp(3, 0, 3, 0)(srch, dsth, *eat, veB,
                                  tbl3a.reshape(-1))
    outsB = _edge_exp(3, 0, 3, 3)(srch, dsth, *eat, veB,
                                  tbl3b.reshape(-1))
    exe3 = list(outsA[:3]) + list(outsB[:3])
    denE3 = jnp.concatenate([_sum32(outsA[3].reshape(NW, NP, 4), 4)[:, 0:3],
                             _sum32(outsB[3].reshape(NW, NP, 4), 4)[:, 0:3]],
                            axis=1)
    msgs3 = [_spmm3(srch, dsth, exe3[2 * vh], exe3[2 * vh + 1],
                    xh3[vh], zr128) for vh in range(3)]
    msg3 = jnp.stack(msgs3, axis=1)
    asn3 = jnp.concatenate([aa3[:, 0:3], aa3[:, 8:11]], axis=1)
    adn3 = jnp.concatenate([aa3[:, 3:6], aa3[:, 11:14]], axis=1)
    nd3 = _nodeprep(asn3, adn3, sumA3[:, 0:6], deg, denE3)
    return _epi3(msg3, xh3, nd3, lin3)


def _spmm3(srch, dsth, exe_a, exe_b, xh_vh, zrow):
    RPT = NP // NS

    @functools.partial(
        pl.kernel,
        out_type=jax.ShapeDtypeStruct((2, NP, 128), F32),
        mesh=_mesh,
        scratch_types=[
            pltpu.VMEM_SHARED((NP, 128), F32),
            pltpu.VMEM((K,), jnp.int32),
            pltpu.VMEM((K,), jnp.int32),
            pltpu.VMEM((K,), F32),
            pltpu.VMEM((K,), F32),
            pltpu.VMEM((K, 128), F32),
            pltpu.SemaphoreType.DMA,
        ],
    )
    def k(srch, dsth, exa, exb, xhv, zr, msg_o, acc, srcv, dstv, wva, wvb,
          rows, sem):
        c = lax.axis_index("c")
        s = lax.axis_index("s")
        pltpu.sync_copy(zr, acc.at[pl.ds(s * RPT, RPT)])
        plsc.subcore_barrier()
        half = NCHUNK // 2

        def chunk(j, carry):
            m = c * half + j * NS + s

            @pl.when(m < (c + 1) * half)
            def _():
                base = m * K
                pltpu.sync_copy(srch.at[pl.ds(base, K)], srcv)
                pltpu.sync_copy(dsth.at[pl.ds(base, K)], dstv)
                pltpu.sync_copy(exa.at[pl.ds(base, K)], wva)
                pltpu.sync_copy(exb.at[pl.ds(base, K)], wvb)
                pltpu.async_copy(xhv.at[srcv], rows, sem).wait()

                def egroup(g, carry2):
                    wsa = wva[pl.ds(g * 16, 16)]
                    wsb = wvb[pl.ds(g * 16, 16)]
                    for kk in range(16):
                        e = g * 16 + kk
                        wa = wsa[kk]
                        wb = wsb[kk]
                        for q in range(8):
                            sl = pl.ds(q * 16, 16)
                            w = wa if q < 4 else wb
                            rows[e, sl] = rows[e, sl] * w
                    return carry2

                lax.fori_loop(0, K // 16, egroup, 0)
                pltpu.sync_copy(rows, acc.at[dstv], add=True)
            return carry

        lax.fori_loop(0, (half + NS - 1) // NS, chunk, 0)
        plsc.subcore_barrier()
        pltpu.sync_copy(acc.at[pl.ds(s * RPT, RPT)],
                        msg_o.at[c, pl.ds(s * RPT, RPT), :])

    return k(srch, dsth, exe_a, exe_b, xh_vh, zrow)


# speedup vs baseline: 17.1411x; 1.2437x over previous
import functools

import jax
import jax.numpy as jnp
from jax import lax
from jax.experimental import pallas as pl
from jax.experimental.pallas import tpu as pltpu
from jax.experimental.pallas import tpu_sc as plsc

F32 = jnp.float32
N = 10000
NP = 10240
E = 320000
BN = 512
NB = NP // BN
NC, NS = 2, 16
NW = NC * NS
K = 128
NCHUNK = E // K
NCHT = (NCHUNK + NW - 1) // NW

_mesh = plsc.VectorSubcoreMesh(core_axis_name="c", subcore_axis_name="s",
                               num_cores=NC, num_subcores=NS)
_sc_params = pltpu.CompilerParams(needs_layout_passes=False)


def _prologue(x, Wv, Wl, bvec, Vsd, VH, lin_blocked):
    F = x.shape[1]
    LW = Wl.shape[1]
    AW = Vsd.shape[1]
    lwb = 128 if lin_blocked else LW

    def body(x_ref, wv_ref, wl_ref, b_ref, vsd_ref, xh_o, lin_o, aa_o):
        vh = pl.program_id(1)
        xb = x_ref[...]
        xh_o[0] = jnp.dot(xb, wv_ref[0], preferred_element_type=F32)
        if lin_blocked:
            lin_o[...] = jnp.dot(xb, wl_ref[...],
                                 preferred_element_type=F32) + b_ref[...]
        else:
            @pl.when(vh == 0)
            def _():
                lin_o[...] = jnp.dot(xb, wl_ref[...],
                                     preferred_element_type=F32) + b_ref[...]

        @pl.when(vh == 0)
        def _():
            aa_o[...] = jnp.dot(xb, vsd_ref[...], preferred_element_type=F32)

    if lin_blocked:
        lin_ix = lambda nb, vh: (nb, vh)
        wl_ix = lambda nb, vh: (0, vh)
    else:
        lin_ix = lambda nb, vh: (nb, 0)
        wl_ix = lambda nb, vh: (0, 0)

    return pl.pallas_call(
        body,
        grid=(NB, VH),
        in_specs=[
            pl.BlockSpec((BN, F), lambda nb, vh: (nb, 0)),
            pl.BlockSpec((1, F, 128), lambda nb, vh: (vh, 0, 0)),
            pl.BlockSpec((F, lwb), wl_ix),
            pl.BlockSpec((1, lwb), wl_ix),
            pl.BlockSpec((F, AW), lambda nb, vh: (0, 0)),
        ],
        out_specs=[
            pl.BlockSpec((1, BN, 128), lambda nb, vh: (vh, nb, 0)),
            pl.BlockSpec((BN, lwb), lin_ix),
            pl.BlockSpec((BN, AW), lambda nb, vh: (nb, 0)),
        ],
        out_shape=[
            jax.ShapeDtypeStruct((VH, NP, 128), F32),
            jax.ShapeDtypeStruct((NP, LW), F32),
            jax.ShapeDtypeStruct((NP, AW), F32),
        ],
    )(x, Wv, Wl, bvec, Vsd)


def _nodeprep(asn, adn, aesum, deg, denE):
    H = asn.shape[1]

    def body(asn_r, adn_r, aes_r, deg_r, den_r, nd_o):
        degc = jnp.maximum(deg_r[...], 1.0)
        s = asn_r[...] + adn_r[...] + aes_r[...] / degc
        al = jnp.where(s > 0, s, 0.2 * s)
        exl = jnp.exp(al)
        den = den_r[...] + exl
        pad = jnp.zeros((BN, 8 - H), F32)
        nd_o[...] = jnp.concatenate([exl, pad, den, pad], axis=1)

    return pl.pallas_call(
        body,
        grid=(NB,),
        in_specs=[pl.BlockSpec((BN, H), lambda nb: (nb, 0))] * 3
        + [pl.BlockSpec((BN, 1), lambda nb: (nb, 0)),
           pl.BlockSpec((BN, H), lambda nb: (nb, 0))],
        out_specs=pl.BlockSpec((BN, 16), lambda nb: (nb, 0)),
        out_shape=jax.ShapeDtypeStruct((NP, 16), F32),
    )(asn, adn, aesum, deg, denE)


def _sum32(acc, C):
    def body(a_ref, o_ref):
        o_ref[...] = jnp.sum(a_ref[...], axis=0)

    return pl.pallas_call(
        body,
        grid=(NB,),
        in_specs=[pl.BlockSpec((NW, BN, C), lambda nb: (0, nb, 0))],
        out_specs=pl.BlockSpec((BN, C), lambda nb: (nb, 0)),
        out_shape=jax.ShapeDtypeStruct((NP, C), F32),
    )(acc)


def _epi12(msg, xh, nd, lin):
    def body(msg_ref, xh_ref, nd_ref, lin_ref, o_ref):
        vh = pl.program_id(1)
        h = vh // 2
        nd_blk = nd_ref[...]
        li = lax.broadcasted_iota(jnp.int32, (1, 16), 1)
        exl = jnp.sum(jnp.where(li == h, nd_blk, 0.0), axis=1, keepdims=True)
        den = jnp.sum(jnp.where(li == 8 + h, nd_blk, 0.0), axis=1,
                      keepdims=True)
        m = msg_ref[0, 0] + msg_ref[1, 0]
        v = (m + exl * xh_ref[0]) / den + lin_ref[...]
        o_ref[...] = jnp.where(v > 0, v, jnp.exp(jnp.minimum(v, 0.0)) - 1.0)

    return pl.pallas_call(
        body,
        grid=(NB, 8),
        in_specs=[
            pl.BlockSpec((2, 1, BN, 128), lambda nb, vh: (0, vh, nb, 0)),
            pl.BlockSpec((1, BN, 128), lambda nb, vh: (vh, nb, 0)),
            pl.BlockSpec((BN, 16), lambda nb, vh: (nb, 0)),
            pl.BlockSpec((BN, 128), lambda nb, vh: (nb, vh)),
        ],
        out_specs=pl.BlockSpec((BN, 128), lambda nb, vh: (nb, vh)),
        out_shape=jax.ShapeDtypeStruct((NP, 1024), F32),
    )(msg, xh, nd, lin)


def _epi3(msg, xh, nd, lin):
    def body(msg_ref, xh_ref, nd_ref, lin_ref, o_ref):
        acc = jnp.zeros((BN, 64), F32)
        for h in range(6):
            lo = (h % 2) * 64
            m = msg_ref[0, h // 2, :, lo:lo + 64] + \
                msg_ref[1, h // 2, :, lo:lo + 64]
            exl = nd_ref[:, h:h + 1]
            den = nd_ref[:, 8 + h:9 + h]
            acc = acc + (m + exl * xh_ref[h // 2, :, lo:lo + 64]) / den
        o_ref[...] = acc[:, :40] / 6.0 + lin_ref[:, :40]

    return pl.pallas_call(
        body,
        grid=(NB,),
        in_specs=[
            pl.BlockSpec((2, 3, BN, 128), lambda nb: (0, 0, nb, 0)),
            pl.BlockSpec((3, BN, 128), lambda nb: (0, nb, 0)),
            pl.BlockSpec((BN, 16), lambda nb: (nb, 0)),
            pl.BlockSpec((BN, 64), lambda nb: (nb, 0)),
        ],
        out_specs=pl.BlockSpec((BN, 40), lambda nb: (nb, 0)),
        out_shape=jax.ShapeDtypeStruct((N, 40), F32),
    )(msg, xh, nd, lin)


def _zero_vmem(ref, nslices):
    z = jnp.zeros((16,), F32)

    def zbody(i, carry):
        for t in range(16):
            ref[pl.ds((i * 16 + t) * 16, 16)] = z
        return carry

    lax.fori_loop(0, nslices // 16, zbody, 0)


def _edge_accum(cols):
    @functools.partial(
        pl.kernel,
        out_type=jax.ShapeDtypeStruct((NW, NP * 8), F32),
        mesh=_mesh,
        compiler_params=_sc_params,
        scratch_types=[
            pltpu.VMEM((NP * 8,), F32),
            pltpu.VMEM((K,), jnp.int32),
            pltpu.VMEM((6, K), F32),
            pltpu.VMEM((6, 16), F32),
        ],
    )
    def k(dsth, ea0, ea1, ea2, ea3, ea4, ea5, vek, acc_o, acc, dstv, eabuf,
          vebuf):
        c = lax.axis_index("c")
        s = lax.axis_index("s")
        wid = c * NS + s
        iota = lax.iota(jnp.int32, 16)
        _zero_vmem(acc, NP * 8 // 16)
        pltpu.sync_copy(vek, vebuf)
        eas = (ea0, ea1, ea2, ea3, ea4, ea5)

        def chunk(j, carry):
            m = j * NW + wid

            @pl.when(m < NCHUNK)
            def _():
                base = m * K
                pltpu.sync_copy(dsth.at[pl.ds(base, K)], dstv)
                for kk in range(6):
                    pltpu.sync_copy(eas[kk].at[pl.ds(base, K)],
                                    eabuf.at[kk])
                for g in range(K // 16):
                    nid = dstv[pl.ds(g * 16, 16)]
                    eak = [eabuf[kk, pl.ds(g * 16, 16)] for kk in range(6)]
                    for col, hv in cols:
                        if hv is None:
                            val = jnp.full((16,), 1.0, F32)
                        else:
                            ver = [vebuf[kk, :] for kk in range(6)]
                            val = eak[0] * ver[0][hv]
                            for kk in range(1, 6):
                                val = val + eak[kk] * ver[kk][hv]
                        plsc.addupdate_scatter(
                            acc, [nid * 8 + col], val)
            return carry

        lax.fori_loop(0, NCHT, chunk, 0)
        pltpu.sync_copy(acc, acc_o.at[wid])

    return k


def _edge_exp(Ht, asn_col, adn_col, ve_col0):
    @functools.partial(
        pl.kernel,
        out_type=[jax.ShapeDtypeStruct((E,), F32) for _ in range(Ht)]
        + [jax.ShapeDtypeStruct((NW, NP * 4), F32)],
        mesh=_mesh,
        compiler_params=_sc_params,
        scratch_types=[
            pltpu.VMEM((NP * 8,), F32),
            pltpu.VMEM((NP * 4,), F32),
            pltpu.VMEM((K,), jnp.int32),
            pltpu.VMEM((K,), jnp.int32),
            pltpu.VMEM((6, K), F32),
            pltpu.VMEM((6, 16), F32),
            pltpu.VMEM((8, K), F32),
        ],
    )
    def k(srch, dsth, ea0, ea1, ea2, ea3, ea4, ea5, vek, tblh, *rest):
        exe_os = rest[:Ht]
        acc_o = rest[Ht]
        tblv, acc, srcv, dstv, eabuf, vebuf, ebuf = rest[Ht + 1:Ht + 8]
        c = lax.axis_index("c")
        s = lax.axis_index("s")
        wid = c * NS + s
        _zero_vmem(acc, NP * 4 // 16)
        pltpu.sync_copy(vek, vebuf)
        pltpu.sync_copy(tblh, tblv)
        eas = (ea0, ea1, ea2, ea3, ea4, ea5)

        def chunk(j, carry):
            m = j * NW + wid

            @pl.when(m < NCHUNK)
            def _():
                base = m * K
                pltpu.sync_copy(srch.at[pl.ds(base, K)], srcv)
                pltpu.sync_copy(dsth.at[pl.ds(base, K)], dstv)
                for kk in range(6):
                    pltpu.sync_copy(eas[kk].at[pl.ds(base, K)],
                                    eabuf.at[kk])
                for g in range(K // 16):
                    nid_s = srcv[pl.ds(g * 16, 16)]
                    nid_d = dstv[pl.ds(g * 16, 16)]
                    eak = [eabuf[kk, pl.ds(g * 16, 16)] for kk in range(6)]
                    ver = [vebuf[kk, :] for kk in range(6)]
                    for h in range(Ht):
                        asn = plsc.load_gather(
                            tblv, [nid_s * 8 + (asn_col + h)])
                        adn = plsc.load_gather(
                            tblv, [nid_d * 8 + (adn_col + h)])
                        aee = eak[0] * ver[0][ve_col0 + h]
                        for kk in range(1, 6):
                            aee = aee + eak[kk] * ver[kk][ve_col0 + h]
                        aa = asn + adn + aee
                        al = jnp.where(aa > 0, aa, aa * 0.2)
                        ex = jnp.exp(al)
                        ebuf[h, pl.ds(g * 16, 16)] = ex
                        plsc.addupdate_scatter(acc, [nid_d * 4 + h], ex)
                for h in range(Ht):
                    pltpu.sync_copy(ebuf.at[h],
                                    exe_os[h].at[pl.ds(base, K)])
            return carry

        lax.fori_loop(0, NCHT, chunk, 0)
        pltpu.sync_copy(acc, acc_o.at[wid])

    return k


def _spmm_db(paired):
    RPT = NP // NS
    half = NCHUNK // 2
    nwv = 2 if paired else 1

    scratch = [pltpu.VMEM_SHARED((NP, 128), F32)]
    for _ in range(2):
        scratch += [pltpu.VMEM((K,), jnp.int32), pltpu.VMEM((K,), jnp.int32)]
        scratch += [pltpu.VMEM((K,), F32)] * nwv
        scratch += [pltpu.VMEM((K, 128), F32)]
        scratch += [pltpu.SemaphoreType.DMA]

    @functools.partial(
        pl.kernel,
        out_type=jax.ShapeDtypeStruct((2, NP, 128), F32),
        mesh=_mesh,
        scratch_types=scratch,
    )
    def k(*args):
        if paired:
            srch, dsth, exa, exb, xhv, zr, msg_o = args[:7]
            rest = args[7:]
        else:
            srch, dsth, exa, xhv, zr, msg_o = args[:6]
            exb = None
            rest = args[6:]
        acc = rest[0]
        per = 4 + nwv
        bufs = []
        for b in range(2):
            grp = rest[1 + b * per:1 + (b + 1) * per]
            bufs.append(grp)
        c = lax.axis_index("c")
        s = lax.axis_index("s")
        pltpu.sync_copy(zr, acc.at[pl.ds(s * RPT, RPT)])
        plsc.subcore_barrier()
        lim = (c + 1) * half
        mbase = c * half + s

        def load(grp, m):
            base = m * K
            pltpu.sync_copy(srch.at[pl.ds(base, K)], grp[0])
            pltpu.sync_copy(dsth.at[pl.ds(base, K)], grp[1])
            pltpu.sync_copy(exa.at[pl.ds(base, K)], grp[2])
            if paired:
                pltpu.sync_copy(exb.at[pl.ds(base, K)], grp[3])
            pltpu.async_copy(xhv.at[grp[0]], grp[per - 2], grp[per - 1])

        def process(grp):
            rows = grp[per - 2]
            pltpu.make_async_copy(xhv.at[grp[0]], rows, grp[per - 1]).wait()

            def egroup(g, carry2):
                wsl = grp[2][pl.ds(g * 16, 16)]
                if paired:
                    wslb = grp[3][pl.ds(g * 16, 16)]
                for kk in range(16):
                    e = g * 16 + kk
                    w = wsl[kk]
                    if paired:
                        wb = wslb[kk]
                    for q in range(8):
                        sl = pl.ds(q * 16, 16)
                        wq = wb if (paired and q >= 4) else w
                        rows[e, sl] = rows[e, sl] * wq
                return carry2

            lax.fori_loop(0, K // 16, egroup, 0)
            pltpu.sync_copy(rows, acc.at[grp[1]], add=True)

        load(bufs[0], mbase)

        def pair_body(jj, carry):
            mA = mbase + (2 * jj) * NS
            mB = mbase + (2 * jj + 1) * NS
            mC = mbase + (2 * jj + 2) * NS

            @pl.when(mB < lim)
            def _():
                load(bufs[1], mB)

            @pl.when(mA < lim)
            def _():
                process(bufs[0])

            @pl.when(mC < lim)
            def _():
                load(bufs[0], mC)

            @pl.when(mB < lim)
            def _():
                process(bufs[1])

            return carry

        lax.fori_loop(0, (half // NS + 2) // 2, pair_body, 0)
        plsc.subcore_barrier()
        pltpu.sync_copy(acc.at[pl.ds(s * RPT, RPT)],
                        msg_o.at[c, pl.ds(s * RPT, RPT), :])

    return k


def _fold(W, att):
    F = W.shape[0]
    H, C = att.shape
    return (W.reshape(F, H, C) * att[None]).sum(-1)


def _vep(We, ae_, col0):
    out = jnp.zeros((6, 16), F32)
    return out.at[:, col0:col0 + ae_.shape[0]].set(_fold(We, ae_))


def kernel(x, edge_index, edge_attr, W1, We1, as1, ad1, ae1, b1, Wl1, bl1,
           W2, We2, as2, ad2, ae2, b2, Wl2, bl2,
           W3, We3, as3, ad3, ae3, b3, Wl3, bl3):
    srch = edge_index[0]
    dsth = edge_index[1]
    eat = [edge_attr[:, kk] for kk in range(6)]
    zr128 = jnp.zeros((NP // NS, 128), F32)

    veA = jnp.zeros((6, 16), F32)
    veA = veA.at[:, 0:4].set(_fold(We1, ae1)).at[:, 4:8].set(_fold(We2, ae2))
    veB = jnp.zeros((6, 16), F32)
    veB = veB.at[:, 0:6].set(_fold(We3, ae3))

    accA12 = _edge_accum([(h, h) for h in range(8)])(dsth, *eat, veA)
    accA3 = _edge_accum([(h, h) for h in range(6)] + [(7, None)])(
        dsth, *eat, veB)
    sumA12 = _sum32(accA12.reshape(NW, NP, 8), 8)
    sumA3 = _sum32(accA3.reshape(NW, NP, 8), 8)
    deg = sumA3[:, 7:8]

    def layer12(x_in, W, Wl, b, bl, as_, ad_, ve_col0):
        F = W.shape[0]
        bias = (b + bl).reshape(1, 1024)
        vs = _fold(W, as_)
        vd = _fold(W, ad_)
        vsd = jnp.concatenate([vs, vd], axis=1)
        Wv = W.reshape(F, 8, 128).transpose(1, 0, 2)
        xh, lin, aa = _prologue(x_in, Wv, Wl, bias, vsd, 8, True)
        veK = veA
        outs = _edge_exp(4, 0, 4, ve_col0)(srch, dsth, *eat, veK,
                                           aa.reshape(-1))
        exes, accB = outs[:4], outs[4]
        denE = _sum32(accB.reshape(NW, NP, 4), 4)
        msgs = []
        for vh in range(8):
            xh_vh = xh[vh]
            ex_vh = exes[vh // 2]
            msgs.append(_spmm_db(False)(srch, dsth, ex_vh, xh_vh, zr128))
        msg = jnp.stack(msgs, axis=1)
        nd = _nodeprep(aa[:, 0:4], aa[:, 4:8], sumA12[:, ve_col0:ve_col0 + 4],
                       deg, denE)
        return _epi12(msg, xh, nd, lin)

    h1 = layer12(x, W1, Wl1, b1, bl1, as1, ad1, 0)
    h2 = layer12(h1, W2, Wl2, b2, bl2, as2, ad2, 4)

    W3v = jnp.pad(W3.reshape(1024, 6, 40),
                  ((0, 0), (0, 0), (0, 24))).reshape(1024, 3, 128)
    W3v = W3v.transpose(1, 0, 2)
    Wl3p = jnp.pad(Wl3, ((0, 0), (0, 24)))
    bias3 = jnp.pad(b3 + bl3, (0, 24)).reshape(1, 64)
    vs3 = _fold(W3, as3)
    vd3 = _fold(W3, ad3)
    vsd3 = jnp.concatenate([vs3[:, 0:3], vd3[:, 0:3], jnp.zeros_like(vs3[:, :1]), jnp.zeros_like(vs3[:, :1]),
                            vs3[:, 3:6], vd3[:, 3:6], jnp.zeros_like(vs3[:, :1]), jnp.zeros_like(vs3[:, :1])], axis=1)
    xh3, lin3, aa3 = _prologue(h2, W3v, Wl3p, bias3, vsd3, 3, False)
    tbl3a = aa3[:, 0:8]
    tbl3b = aa3[:, 8:16]
    outsA = _edge_exp(3, 0, 3, 0)(srch, dsth, *eat, veB,
                                  tbl3a.reshape(-1))
    outsB = _edge_exp(3, 0, 3, 3)(srch, dsth, *eat, veB,
                                  tbl3b.reshape(-1))
    exe3 = list(outsA[:3]) + list(outsB[:3])
    denE3 = jnp.concatenate([_sum32(outsA[3].reshape(NW, NP, 4), 4)[:, 0:3],
                             _sum32(outsB[3].reshape(NW, NP, 4), 4)[:, 0:3]],
                            axis=1)
    msgs3 = [_spmm_db(True)(srch, dsth, exe3[2 * vh], exe3[2 * vh + 1],
                            xh3[vh], zr128) for vh in range(3)]
    msg3 = jnp.stack(msgs3, axis=1)
    asn3 = jnp.concatenate([aa3[:, 0:3], aa3[:, 8:11]], axis=1)
    adn3 = jnp.concatenate([aa3[:, 3:6], aa3[:, 11:14]], axis=1)
    nd3 = _nodeprep(asn3, adn3, sumA3[:, 0:6], deg, denE3)
    return _epi3(msg3, xh3, nd3, lin3)


# speedup vs baseline: 22.7517x; 1.3273x over previous
import functools

import jax
import jax.numpy as jnp
from jax import lax
from jax.experimental import pallas as pl
from jax.experimental.pallas import tpu as pltpu
from jax.experimental.pallas import tpu_sc as plsc

F32 = jnp.float32
N = 10000
NP = 10240
E = 320000
BN = 512
NB = NP // BN
NC, NS = 2, 16
NW = NC * NS
K = 128
NCHUNK = E // K
NCHT = (NCHUNK + NW - 1) // NW

_mesh = plsc.VectorSubcoreMesh(core_axis_name="c", subcore_axis_name="s",
                               num_cores=NC, num_subcores=NS)
_sc_params = pltpu.CompilerParams(needs_layout_passes=False)


def _prologue(x, Wv, Wl, bvec, Vsd, VH, lin_blocked):
    F = x.shape[1]
    LW = Wl.shape[1]
    AW = Vsd.shape[1]
    lwb = 128 if lin_blocked else LW

    def body(x_ref, wv_ref, wl_ref, b_ref, vsd_ref, xh_o, lin_o, aa_o):
        vh = pl.program_id(1)
        xb = x_ref[...]
        xh_o[0] = jnp.dot(xb, wv_ref[0], preferred_element_type=F32)
        if lin_blocked:
            lin_o[...] = jnp.dot(xb, wl_ref[...],
                                 preferred_element_type=F32) + b_ref[...]
        else:
            @pl.when(vh == 0)
            def _():
                lin_o[...] = jnp.dot(xb, wl_ref[...],
                                     preferred_element_type=F32) + b_ref[...]

        @pl.when(vh == 0)
        def _():
            aa_o[...] = jnp.dot(xb, vsd_ref[...], preferred_element_type=F32)

    if lin_blocked:
        lin_ix = lambda nb, vh: (nb, vh)
        wl_ix = lambda nb, vh: (0, vh)
    else:
        lin_ix = lambda nb, vh: (nb, 0)
        wl_ix = lambda nb, vh: (0, 0)

    return pl.pallas_call(
        body,
        grid=(NB, VH),
        in_specs=[
            pl.BlockSpec((BN, F), lambda nb, vh: (nb, 0)),
            pl.BlockSpec((1, F, 128), lambda nb, vh: (vh, 0, 0)),
            pl.BlockSpec((F, lwb), wl_ix),
            pl.BlockSpec((1, lwb), wl_ix),
            pl.BlockSpec((F, AW), lambda nb, vh: (0, 0)),
        ],
        out_specs=[
            pl.BlockSpec((1, BN, 128), lambda nb, vh: (vh, nb, 0)),
            pl.BlockSpec((BN, lwb), lin_ix),
            pl.BlockSpec((BN, AW), lambda nb, vh: (nb, 0)),
        ],
        out_shape=[
            jax.ShapeDtypeStruct((VH, NP, 128), F32),
            jax.ShapeDtypeStruct((NP, LW), F32),
            jax.ShapeDtypeStruct((NP, AW), F32),
        ],
    )(x, Wv, Wl, bvec, Vsd)


def _nodeprep(asn, adn, aesum, deg, denE):
    H = asn.shape[1]

    def body(asn_r, adn_r, aes_r, deg_r, den_r, nd_o):
        degc = jnp.maximum(deg_r[...], 1.0)
        s = asn_r[...] + adn_r[...] + aes_r[...] / degc
        al = jnp.where(s > 0, s, 0.2 * s)
        exl = jnp.exp(al)
        den = den_r[...] + exl
        pad = jnp.zeros((BN, 8 - H), F32)
        nd_o[...] = jnp.concatenate([exl, pad, den, pad], axis=1)

    return pl.pallas_call(
        body,
        grid=(NB,),
        in_specs=[pl.BlockSpec((BN, H), lambda nb: (nb, 0))] * 3
        + [pl.BlockSpec((BN, 1), lambda nb: (nb, 0)),
           pl.BlockSpec((BN, H), lambda nb: (nb, 0))],
        out_specs=pl.BlockSpec((BN, 16), lambda nb: (nb, 0)),
        out_shape=jax.ShapeDtypeStruct((NP, 16), F32),
    )(asn, adn, aesum, deg, denE)


def _sum32(acc, C):
    def body(a_ref, o_ref):
        o_ref[...] = jnp.sum(a_ref[...], axis=0)

    return pl.pallas_call(
        body,
        grid=(NB,),
        in_specs=[pl.BlockSpec((NW, BN, C), lambda nb: (0, nb, 0))],
        out_specs=pl.BlockSpec((BN, C), lambda nb: (nb, 0)),
        out_shape=jax.ShapeDtypeStruct((NP, C), F32),
    )(acc)


def _epi12(msg, xh, nd, lin):
    def body(msg_ref, xh_ref, nd_ref, lin_ref, o_ref):
        vh = pl.program_id(1)
        h = vh // 2
        nd_blk = nd_ref[...]
        li = lax.broadcasted_iota(jnp.int32, (1, 16), 1)
        exl = jnp.sum(jnp.where(li == h, nd_blk, 0.0), axis=1, keepdims=True)
        den = jnp.sum(jnp.where(li == 8 + h, nd_blk, 0.0), axis=1,
                      keepdims=True)
        m = msg_ref[0, 0] + msg_ref[1, 0]
        v = (m + exl * xh_ref[0]) / den + lin_ref[...]
        o_ref[...] = jnp.where(v > 0, v, jnp.exp(jnp.minimum(v, 0.0)) - 1.0)

    return pl.pallas_call(
        body,
        grid=(NB, 8),
        in_specs=[
            pl.BlockSpec((2, 1, BN, 128), lambda nb, vh: (0, vh, nb, 0)),
            pl.BlockSpec((1, BN, 128), lambda nb, vh: (vh, nb, 0)),
            pl.BlockSpec((BN, 16), lambda nb, vh: (nb, 0)),
            pl.BlockSpec((BN, 128), lambda nb, vh: (nb, vh)),
        ],
        out_specs=pl.BlockSpec((BN, 128), lambda nb, vh: (nb, vh)),
        out_shape=jax.ShapeDtypeStruct((NP, 1024), F32),
    )(msg, xh, nd, lin)


def _epi3(msg, xh, nd, lin):
    def body(msg_ref, xh_ref, nd_ref, lin_ref, o_ref):
        acc = jnp.zeros((BN, 64), F32)
        for h in range(6):
            lo = (h % 2) * 64
            m = msg_ref[0, h // 2, :, lo:lo + 64] + \
                msg_ref[1, h // 2, :, lo:lo + 64]
            exl = nd_ref[:, h:h + 1]
            den = nd_ref[:, 8 + h:9 + h]
            acc = acc + (m + exl * xh_ref[h // 2, :, lo:lo + 64]) / den
        o_ref[...] = acc[:, :40] / 6.0 + lin_ref[:, :40]

    return pl.pallas_call(
        body,
        grid=(NB,),
        in_specs=[
            pl.BlockSpec((2, 3, BN, 128), lambda nb: (0, 0, nb, 0)),
            pl.BlockSpec((3, BN, 128), lambda nb: (0, nb, 0)),
            pl.BlockSpec((BN, 16), lambda nb: (nb, 0)),
            pl.BlockSpec((BN, 64), lambda nb: (nb, 0)),
        ],
        out_specs=pl.BlockSpec((BN, 40), lambda nb: (nb, 0)),
        out_shape=jax.ShapeDtypeStruct((N, 40), F32),
    )(msg, xh, nd, lin)


def _zero_vmem(ref, nslices):
    z = jnp.zeros((16,), F32)

    def zbody(i, carry):
        for t in range(16):
            ref[pl.ds((i * 16 + t) * 16, 16)] = z
        return carry

    lax.fori_loop(0, nslices // 16, zbody, 0)


def _edge_accum(cols):
    @functools.partial(
        pl.kernel,
        out_type=jax.ShapeDtypeStruct((NW, NP * 8), F32),
        mesh=_mesh,
        compiler_params=_sc_params,
        scratch_types=[
            pltpu.VMEM((NP * 8,), F32),
            pltpu.VMEM((2, K), jnp.int32),
            pltpu.VMEM((6, K), F32),
            pltpu.VMEM((6, 16), F32),
        ],
    )
    def k(ei3, ea3d, vek, acc_o, acc, eibuf, eabuf, vebuf):
        c = lax.axis_index("c")
        s = lax.axis_index("s")
        wid = c * NS + s
        iota = lax.iota(jnp.int32, 16)
        _zero_vmem(acc, NP * 8 // 16)
        pltpu.sync_copy(vek, vebuf)

        def chunk(j, carry):
            m = j * NW + wid

            @pl.when(m < NCHUNK)
            def _():
                pltpu.sync_copy(ei3.at[m], eibuf)
                pltpu.sync_copy(ea3d.at[m], eabuf)
                for g in range(K // 16):
                    nid = eibuf[1, pl.ds(g * 16, 16)]
                    eak = [eabuf[kk, pl.ds(g * 16, 16)] for kk in range(6)]
                    for col, hv in cols:
                        if hv is None:
                            val = jnp.full((16,), 1.0, F32)
                        else:
                            ver = [vebuf[kk, :] for kk in range(6)]
                            val = eak[0] * ver[0][hv]
                            for kk in range(1, 6):
                                val = val + eak[kk] * ver[kk][hv]
                        plsc.addupdate_scatter(
                            acc, [nid * 8 + col], val)
            return carry

        lax.fori_loop(0, NCHT, chunk, 0)
        pltpu.sync_copy(acc, acc_o.at[wid])

    return k


def _edge_exp(Ht, asn_col, adn_col, ve_col0):
    @functools.partial(
        pl.kernel,
        out_type=[jax.ShapeDtypeStruct((NCHUNK, 8, K), F32),
                  jax.ShapeDtypeStruct((NW, NP * 4), F32)],
        mesh=_mesh,
        compiler_params=_sc_params,
        scratch_types=[
            pltpu.VMEM((NP * 8,), F32),
            pltpu.VMEM((NP * 4,), F32),
            pltpu.VMEM((2, K), jnp.int32),
            pltpu.VMEM((6, K), F32),
            pltpu.VMEM((6, 16), F32),
            pltpu.VMEM((8, K), F32),
        ],
    )
    def k(ei3, ea3d, vek, tblh, exe_o, acc_o, tblv, acc, eibuf, eabuf,
          vebuf, ebuf):
        c = lax.axis_index("c")
        s = lax.axis_index("s")
        wid = c * NS + s
        _zero_vmem(acc, NP * 4 // 16)
        pltpu.sync_copy(vek, vebuf)
        pltpu.sync_copy(tblh, tblv)

        def chunk(j, carry):
            m = j * NW + wid

            @pl.when(m < NCHUNK)
            def _():
                pltpu.sync_copy(ei3.at[m], eibuf)
                pltpu.sync_copy(ea3d.at[m], eabuf)
                for g in range(K // 16):
                    nid_s = eibuf[0, pl.ds(g * 16, 16)]
                    nid_d = eibuf[1, pl.ds(g * 16, 16)]
                    eak = [eabuf[kk, pl.ds(g * 16, 16)] for kk in range(6)]
                    ver = [vebuf[kk, :] for kk in range(6)]
                    for h in range(Ht):
                        asn = plsc.load_gather(
                            tblv, [nid_s * 8 + (asn_col + h)])
                        adn = plsc.load_gather(
                            tblv, [nid_d * 8 + (adn_col + h)])
                        aee = eak[0] * ver[0][ve_col0 + h]
                        for kk in range(1, 6):
                            aee = aee + eak[kk] * ver[kk][ve_col0 + h]
                        aa = asn + adn + aee
                        al = jnp.where(aa > 0, aa, aa * 0.2)
                        ex = jnp.exp(al)
                        ebuf[h, pl.ds(g * 16, 16)] = ex
                        plsc.addupdate_scatter(acc, [nid_d * 4 + h], ex)
                pltpu.sync_copy(ebuf, exe_o.at[m])
            return carry

        lax.fori_loop(0, NCHT, chunk, 0)
        pltpu.sync_copy(acc, acc_o.at[wid])

    return k


def _spmm_db(paired, ha, hb=0):
    RPT = NP // NS
    half = NCHUNK // 2
    nwv = 2 if paired else 1

    scratch = [pltpu.VMEM_SHARED((NP, 128), F32)]
    for _ in range(2):
        scratch += [pltpu.VMEM((2, K), jnp.int32)]
        scratch += [pltpu.VMEM((K,), F32)] * nwv
        scratch += [pltpu.VMEM((K, 128), F32)]
        scratch += [pltpu.SemaphoreType.DMA]

    @functools.partial(
        pl.kernel,
        out_type=jax.ShapeDtypeStruct((2, NP, 128), F32),
        mesh=_mesh,
        scratch_types=scratch,
    )
    def k(*args):
        if paired:
            ei3, exa, exb, xhv, zr, msg_o = args[:6]
            rest = args[6:]
        else:
            ei3, exa, xhv, zr, msg_o = args[:5]
            exb = None
            rest = args[5:]
        acc = rest[0]
        per = 3 + nwv
        bufs = [rest[1 + b * per:1 + (b + 1) * per] for b in range(2)]
        c = lax.axis_index("c")
        s = lax.axis_index("s")
        pltpu.sync_copy(zr, acc.at[pl.ds(s * RPT, RPT)])
        plsc.subcore_barrier()
        lim = (c + 1) * half
        mbase = c * half + s

        def load(grp, m):
            pltpu.sync_copy(ei3.at[m], grp[0])
            pltpu.sync_copy(exa.at[m, ha], grp[1])
            if paired:
                pltpu.sync_copy(exb.at[m, hb], grp[2])
            pltpu.async_copy(xhv.at[grp[0].at[0]], grp[per - 2],
                             grp[per - 1])

        def process(grp):
            rows = grp[per - 2]
            pltpu.make_async_copy(xhv.at[grp[0].at[0]], rows,
                                  grp[per - 1]).wait()

            def egroup(g, carry2):
                wsl = grp[1][pl.ds(g * 16, 16)]
                if paired:
                    wslb = grp[2][pl.ds(g * 16, 16)]
                for kk in range(16):
                    e = g * 16 + kk
                    w = wsl[kk]
                    if paired:
                        wb = wslb[kk]
                    for q in range(8):
                        sl = pl.ds(q * 16, 16)
                        wq = wb if (paired and q >= 4) else w
                        rows[e, sl] = rows[e, sl] * wq
                return carry2

            lax.fori_loop(0, K // 16, egroup, 0)
            pltpu.sync_copy(rows, acc.at[grp[0].at[1]], add=True)

        load(bufs[0], mbase)

        def pair_body(jj, carry):
            mA = mbase + (2 * jj) * NS
            mB = mbase + (2 * jj + 1) * NS
            mC = mbase + (2 * jj + 2) * NS

            @pl.when(mB < lim)
            def _():
                load(bufs[1], mB)

            @pl.when(mA < lim)
            def _():
                process(bufs[0])

            @pl.when(mC < lim)
            def _():
                load(bufs[0], mC)

            @pl.when(mB < lim)
            def _():
                process(bufs[1])

            return carry

        lax.fori_loop(0, (half // NS + 2) // 2, pair_body, 0)
        plsc.subcore_barrier()
        pltpu.sync_copy(acc.at[pl.ds(s * RPT, RPT)],
                        msg_o.at[c, pl.ds(s * RPT, RPT), :])

    return k


def _fold(W, att):
    F = W.shape[0]
    H, C = att.shape
    return (W.reshape(F, H, C) * att[None]).sum(-1)


def _vep(We, ae_, col0):
    out = jnp.zeros((6, 16), F32)
    return out.at[:, col0:col0 + ae_.shape[0]].set(_fold(We, ae_))


def kernel(x, edge_index, edge_attr, W1, We1, as1, ad1, ae1, b1, Wl1, bl1,
           W2, We2, as2, ad2, ae2, b2, Wl2, bl2,
           W3, We3, as3, ad3, ae3, b3, Wl3, bl3):
    ei3 = edge_index.reshape(2, NCHUNK, K).transpose(1, 0, 2)
    ea3d = edge_attr.reshape(NCHUNK, K, 6).transpose(0, 2, 1)
    zr128 = jnp.zeros((NP // NS, 128), F32)

    veA = jnp.zeros((6, 16), F32)
    veA = veA.at[:, 0:4].set(_fold(We1, ae1)).at[:, 4:8].set(_fold(We2, ae2))
    veB = jnp.zeros((6, 16), F32)
    veB = veB.at[:, 0:6].set(_fold(We3, ae3))

    accA12 = _edge_accum([(h, h) for h in range(8)])(ei3, ea3d, veA)
    accA3 = _edge_accum([(h, h) for h in range(6)] + [(7, None)])(
        ei3, ea3d, veB)
    sumA12 = _sum32(accA12.reshape(NW, NP, 8), 8)
    sumA3 = _sum32(accA3.reshape(NW, NP, 8), 8)
    deg = sumA3[:, 7:8]

    def layer12(x_in, W, Wl, b, bl, as_, ad_, ve_col0):
        F = W.shape[0]
        bias = (b + bl).reshape(1, 1024)
        vs = _fold(W, as_)
        vd = _fold(W, ad_)
        vsd = jnp.concatenate([vs, vd], axis=1)
        Wv = W.reshape(F, 8, 128).transpose(1, 0, 2)
        xh, lin, aa = _prologue(x_in, Wv, Wl, bias, vsd, 8, True)
        veK = veA
        exe3d, accB = _edge_exp(4, 0, 4, ve_col0)(ei3, ea3d, veK,
                                                  aa.reshape(-1))
        denE = _sum32(accB.reshape(NW, NP, 4), 4)
        msgs = []
        for vh in range(8):
            msgs.append(_spmm_db(False, vh // 2)(ei3, exe3d, xh[vh],
                                                 zr128))
        msg = jnp.stack(msgs, axis=1)
        nd = _nodeprep(aa[:, 0:4], aa[:, 4:8], sumA12[:, ve_col0:ve_col0 + 4],
                       deg, denE)
        return _epi12(msg, xh, nd, lin)

    h1 = layer12(x, W1, Wl1, b1, bl1, as1, ad1, 0)
    h2 = layer12(h1, W2, Wl2, b2, bl2, as2, ad2, 4)

    W3v = jnp.pad(W3.reshape(1024, 6, 40),
                  ((0, 0), (0, 0), (0, 24))).reshape(1024, 3, 128)
    W3v = W3v.transpose(1, 0, 2)
    Wl3p = jnp.pad(Wl3, ((0, 0), (0, 24)))
    bias3 = jnp.pad(b3 + bl3, (0, 24)).reshape(1, 64)
    vs3 = _fold(W3, as3)
    vd3 = _fold(W3, ad3)
    vsd3 = jnp.concatenate([vs3[:, 0:3], vd3[:, 0:3], jnp.zeros_like(vs3[:, :1]), jnp.zeros_like(vs3[:, :1]),
                            vs3[:, 3:6], vd3[:, 3:6], jnp.zeros_like(vs3[:, :1]), jnp.zeros_like(vs3[:, :1])], axis=1)
    xh3, lin3, aa3 = _prologue(h2, W3v, Wl3p, bias3, vsd3, 3, False)
    tbl3a = aa3[:, 0:8]
    tbl3b = aa3[:, 8:16]
    exe3dA, accB3a = _edge_exp(3, 0, 3, 0)(ei3, ea3d, veB,
                                           tbl3a.reshape(-1))
    exe3dB, accB3b = _edge_exp(3, 0, 3, 3)(ei3, ea3d, veB,
                                           tbl3b.reshape(-1))
    denE3 = jnp.concatenate([_sum32(accB3a.reshape(NW, NP, 4), 4)[:, 0:3],
                             _sum32(accB3b.reshape(NW, NP, 4), 4)[:, 0:3]],
                            axis=1)
    def exe_for(h):
        return (exe3dA, h) if h < 3 else (exe3dB, h - 3)

    msgs3 = []
    for vh in range(3):
        (ea_arr, hra) = exe_for(2 * vh)
        (eb_arr, hrb) = exe_for(2 * vh + 1)
        msgs3.append(_spmm_db(True, hra, hrb)(ei3, ea_arr, eb_arr,
                                              xh3[vh], zr128))
    msg3 = jnp.stack(msgs3, axis=1)
    asn3 = jnp.concatenate([aa3[:, 0:3], aa3[:, 8:11]], axis=1)
    adn3 = jnp.concatenate([aa3[:, 3:6], aa3[:, 11:14]], axis=1)
    nd3 = _nodeprep(asn3, adn3, sumA3[:, 0:6], deg, denE3)
    return _epi3(msg3, xh3, nd3, lin3)
